# baseline (speedup 1.0000x reference)
import jax
import jax.numpy as jnp
from jax import lax
from jax.experimental import pallas as pl
from jax.experimental.pallas import tpu as pltpu
from jax.experimental.pallas import tpu_sc as plsc

_B = 16384
_H = 50
_GL = 8
_NC = 2
_NS = 16
_NW = _NC * _NS
_RPW = _B // _NW
_R = 16
_NCHUNK = _RPW // _R

_NH = _R * _H
_NG = _R * _GL

_HPIECES = [(0, 8), (8, 8), (16, 8), (24, 8), (32, 8), (40, 8), (48, 2)]

_OUT_D = 374


def _blk(a):
    return a.reshape(_NW, _NCHUNK, _R, a.shape[1]) \
            .transpose(0, 1, 3, 2).reshape(-1)


def _sc_body(sid, city, lang, agec, gend, rvia, sst, ssn, stype,
             rit, expd, agen, ten, yy, slen,
             qh, qc, ah, ac, am, gid, glen,
             t_song, t_art, t_genre, t_city, t_gr, t_agec, t_lang, t_ctx,
             out,
             idx_s, w_s, rows_s, idx_a, w_ac, w_am, rows_a,
             idx_g, w_g, rows_g,
             idx_sml, scal_buf,
             rows_sid, buf_city, buf_lang, buf_agec, buf_gr, buf_ctx,
             acc_gen, out_buf,
             sem_a, sem_b, sem_g, sem_s, sem_o):
    wid = lax.axis_index("s") * _NC + lax.axis_index("c")
    base = wid * _RPW
    lane = lax.iota(jnp.int32, 16)
    gdn = lax.GatherDimensionNumbers(offset_dims=(), collapsed_slice_dims=(0,),
                                     start_index_map=(0,))

    def bcastl(vec, r):
        idx = jnp.bitwise_and(lane * 0 + r, 15)[:, None]
        return lax.gather(vec, idx, gdn, (1,),
                          mode=lax.GatherScatterMode.PROMISE_IN_BOUNDS)

    lanef = lane.astype(jnp.float32)
    m0 = jnp.maximum(1.0 - lanef, 0.0)
    m1 = jnp.maximum(1.0 - jnp.abs(lanef - 1.0), 0.0)

    def lane0(vec):
        idx = jnp.bitwise_and(lane, 0)[:, None]
        return lax.gather(vec, idx, gdn, (1,),
                          mode=lax.GatherScatterMode.PROMISE_IN_BOUNDS)

    def wsum_into(rows, wlist, nh, d, acc_ref, cols, stride):
        nj = d // 16
        nw = len(wlist)

        def row_body(r, carry):
            def h_body(h, accs):
                p = h * _R + r
                rs = [rows[p, pl.ds(j * 16, 16)] for j in range(nj)]
                ws = [bcastl(w[pl.ds(h * _R, 16)], r) for w in wlist]
                return tuple(accs[i * nj + j] + rs[j] * ws[i]
                             for i in range(nw) for j in range(nj))

            init = tuple(jnp.zeros((16,), jnp.float32)
                         for _ in range(nw * nj))
            accs = lax.fori_loop(0, nh, h_body, init)
            for i in range(nw):
                for j in range(nj):
                    acc_ref[pl.ds(r * stride + cols[i] + j * 16, 16)] = \
                        accs[i * nj + j]
            return carry

        lax.fori_loop(0, _R, row_body, 0)

    def chunk_body(c, carry):
        row0 = base + c * _R
        rows_slice = pl.ds(row0, _R)
        hoff = (wid * _NCHUNK + c) * _NH
        goff = (wid * _NCHUNK + c) * _NG

        pltpu.sync_copy(qh.at[pl.ds(hoff, _NH)], idx_s)
        pltpu.sync_copy(ah.at[pl.ds(hoff, _NH)], idx_a)
        song_cps = [pltpu.async_copy(t_song.at[idx_s.at[pl.ds(o * _R, n * _R)]],
                                     rows_s.at[pl.ds(o * _R, n * _R)], sem_a)
                    for o, n in _HPIECES]
        art_cps = [pltpu.async_copy(t_art.at[idx_a.at[pl.ds(o * _R, n * _R)]],
                                    rows_a.at[pl.ds(o * _R, n * _R)], sem_b)
                   for o, n in _HPIECES]

        idx_srcs = [sid, city, lang, agec, gend, rvia, sst, ssn, stype]
        idx_cps = [pltpu.async_copy(a.at[rows_slice],
                                    idx_sml.at[pl.ds(k * 16, 16)], sem_s)
                   for k, a in enumerate(idx_srcs)]
        scal_srcs = [rit, expd, agen, ten, yy, slen]
        scal_cps = [pltpu.async_copy(a.at[rows_slice],
                                     scal_buf.at[pl.ds(k * 16, 16)], sem_s)
                    for k, a in enumerate(scal_srcs)]

        pltpu.sync_copy(qc.at[pl.ds(hoff, _NH)], w_s)
        pltpu.sync_copy(ac.at[pl.ds(hoff, _NH)], w_ac)
        pltpu.sync_copy(am.at[pl.ds(hoff, _NH)], w_am)

        pltpu.sync_copy(gid.at[pl.ds(goff, _NG)], idx_g)
        pltpu.sync_copy(glen.at[pl.ds(goff, _NG)], w_g)
        genre_cp = pltpu.async_copy(t_genre.at[idx_g], rows_g, sem_g)

        for cp in idx_cps:
            cp.wait()
        cidx_gr = (idx_sml[pl.ds(64, 16)] * 10 + idx_sml[pl.ds(80, 16)])
        cidx_ctx = (idx_sml[pl.ds(96, 16)] * 375
                    + idx_sml[pl.ds(112, 16)] * 15
                    + idx_sml[pl.ds(128, 16)])
        sml_cps = [
            pltpu.async_copy(t_song.at[idx_sml.at[pl.ds(0, 16)]],
                             rows_sid, sem_g),
            pltpu.async_copy(t_city.at[idx_sml.at[pl.ds(16, 16)]],
                             buf_city, sem_g),
            pltpu.async_copy(t_lang.at[idx_sml.at[pl.ds(32, 16)]],
                             buf_lang, sem_g),
            pltpu.async_copy(t_agec.at[idx_sml.at[pl.ds(48, 16)]],
                             buf_agec, sem_g),
            pltpu.async_copy(t_gr.at[cidx_gr], buf_gr, sem_g),
            pltpu.async_copy(t_ctx.at[cidx_ctx], buf_ctx, sem_g),
        ]
        genre_cp.wait()
        wsum_into(rows_g, [w_g], _GL, 32, acc_gen, [0], 32)
        for cp in sml_cps:
            cp.wait()
        for cp in scal_cps:
            cp.wait()

        def asm_body(r, carry):
            def pair(ka, kb):
                return (bcastl(scal_buf[pl.ds(ka * 16, 16)], r) * m0
                        + bcastl(scal_buf[pl.ds(kb * 16, 16)], r) * m1)

            o = r * _OUT_D
            out_buf[pl.ds(o + 0, 16)] = buf_city[r, pl.ds(0, 16)]
            out_buf[pl.ds(o + 16, 16)] = buf_gr[r, pl.ds(0, 16)]
            out_buf[pl.ds(o + 32, 16)] = pair(0, 1)
            out_buf[pl.ds(o + 34, 16)] = buf_agec[r, pl.ds(0, 16)]
            out_buf[pl.ds(o + 42, 16)] = pair(2, 3)
            for j in range(4):
                out_buf[pl.ds(o + 236 + j * 16, 16)] = \
                    rows_sid[r, pl.ds(j * 16, 16)]
            out_buf[pl.ds(o + 300, 16)] = buf_lang[r, pl.ds(0, 16)]
            out_buf[pl.ds(o + 316, 16)] = pair(4, 5)
            out_buf[pl.ds(o + 318, 16)] = acc_gen[pl.ds(r * 32, 16)]
            out_buf[pl.ds(o + 334, 16)] = acc_gen[pl.ds(r * 32 + 16, 16)]
            out_buf[pl.ds(o + 350, 16)] = buf_ctx[r, pl.ds(0, 16)]
            out_buf[pl.ds(o + 358, 16)] = buf_ctx[r, pl.ds(8, 16)]
            return carry

        lax.fori_loop(0, _R, asm_body, 0)

        for cp in song_cps:
            cp.wait()
        wsum_into(rows_s, [w_s], _H, 64, out_buf, [44], _OUT_D)
        for cp in art_cps:
            cp.wait()
        wsum_into(rows_a, [w_ac, w_am], _H, 64, out_buf, [108, 172], _OUT_D)

        pltpu.async_copy(out_buf, out.at[pl.ds(row0 * _OUT_D, _R * _OUT_D)],
                         sem_o).wait()
        return carry

    lax.fori_loop(0, _NCHUNK, chunk_body, 0)


@jax.jit
def _run(sid, city, lang, agec, gend, rvia, sst, ssn, stype,
         rit, expd, agen, ten, yy, slen,
         qh, qc, ah, ac, am, gid, glen,
         t_song, t_art, t_genre, t_city, t_gr, t_agec, t_lang, t_ctx):
    mesh = plsc.VectorSubcoreMesh(core_axis_name="c", subcore_axis_name="s")
    k = pl.kernel(
        _sc_body,
        out_type=jax.ShapeDtypeStruct((_B * _OUT_D,), jnp.float32),
        mesh=mesh,
        compiler_params=pltpu.CompilerParams(use_tc_tiling_on_sc=False),
        scratch_types=[
            pltpu.VMEM((_NH,), jnp.int32),
            pltpu.VMEM((_NH,), jnp.float32),
            pltpu.VMEM((_NH, 64), jnp.float32),
            pltpu.VMEM((_NH,), jnp.int32),
            pltpu.VMEM((_NH,), jnp.float32),
            pltpu.VMEM((_NH,), jnp.float32),
            pltpu.VMEM((_NH, 64), jnp.float32),
            pltpu.VMEM((_NG,), jnp.int32),
            pltpu.VMEM((_NG,), jnp.float32),
            pltpu.VMEM((_NG, 32), jnp.float32),
            pltpu.VMEM((144,), jnp.int32),
            pltpu.VMEM((96,), jnp.float32),
            pltpu.VMEM((_R, 64), jnp.float32),
            pltpu.VMEM((_R, 16), jnp.float32),
            pltpu.VMEM((_R, 16), jnp.float32),
            pltpu.VMEM((_R, 16), jnp.float32),
            pltpu.VMEM((_R, 16), jnp.float32),
            pltpu.VMEM((_R, 32), jnp.float32),
            pltpu.VMEM((_R * 32,), jnp.float32),
            pltpu.VMEM((_R * _OUT_D,), jnp.float32),
            pltpu.SemaphoreType.DMA,
            pltpu.SemaphoreType.DMA,
            pltpu.SemaphoreType.DMA,
            pltpu.SemaphoreType.DMA,
            pltpu.SemaphoreType.DMA,
        ],
    )
    return k(sid, city, lang, agec, gend, rvia, sst, ssn, stype,
             rit, expd, agen, ten, yy, slen,
             qh, qc, ah, ac, am, gid, glen,
             t_song, t_art, t_genre, t_city, t_gr, t_agec, t_lang, t_ctx)


def kernel(city, gender, registered_via, msno_age_catg, registration_init_time,
           expiration_date, msno_age_num, msno_tenure,
           msno_pos_query_hist, msno_pos_query_count,
           msno_artist_name_hist, msno_artist_name_count, msno_artist_name_mean,
           song_id, language, song_yy, song_length,
           genre_ids, song_genre_ids_len,
           source_system_tab, source_screen_name, source_type,
           emb_song, emb_artist, emb_genre, emb_city, emb_gender,
           emb_registered_via, emb_msno_age_catg, emb_language,
           emb_sst, emb_ssn, emb_stype):
    t_gr = jnp.concatenate([jnp.repeat(emb_gender, 10, axis=0),
                            jnp.tile(emb_registered_via, (3, 1))], axis=1)
    z = jnp.zeros((10, 25, 15, 8), jnp.float32)
    t_ctx = jnp.concatenate([
        jnp.broadcast_to(emb_sst[:, None, None, :], (10, 25, 15, 8)),
        jnp.broadcast_to(emb_ssn[None, :, None, :], (10, 25, 15, 8)),
        jnp.broadcast_to(emb_stype[None, None, :, :], (10, 25, 15, 8)),
        z], axis=3).reshape(3750, 32)
    t_agec = jnp.pad(emb_msno_age_catg, ((0, 0), (0, 8)))
    flat = _run(song_id, city, language, msno_age_catg,
                gender, registered_via,
                source_system_tab, source_screen_name, source_type,
                registration_init_time, expiration_date,
                msno_age_num, msno_tenure, song_yy, song_length,
                _blk(msno_pos_query_hist), _blk(msno_pos_query_count),
                _blk(msno_artist_name_hist),
                _blk(msno_artist_name_count),
                _blk(msno_artist_name_mean),
                _blk(genre_ids), _blk(song_genre_ids_len),
                emb_song, emb_artist, emb_genre, emb_city, t_gr, t_agec,
                emb_language, t_ctx)
    return flat.reshape(_B, _OUT_D)


# speedup vs baseline: 1.1063x; 1.1063x over previous
import jax
import jax.numpy as jnp
from jax import lax
from jax.experimental import pallas as pl
from jax.experimental.pallas import tpu as pltpu
from jax.experimental.pallas import tpu_sc as plsc

_B = 16384
_H = 50
_GL = 8
_NC = 2
_NS = 16
_NW = _NC * _NS
_RPW = _B // _NW
_R = 16
_NCHUNK = _RPW // _R

_NH = _R * _H
_NG = _R * _GL

_HPIECES = [(0, 128), (128, 128), (256, 128), (384, 128),
            (512, 128), (640, 128), (768, 32)]

_OUT_D = 374


def _sc_body(sid, city, lang, agec, gend, rvia, sst, ssn, stype,
             rit, expd, agen, ten, yy, slen,
             qh, qc, ah, ac, am, gid, glen,
             t_song, t_art, t_genre, t_city, t_gr, t_agec, t_lang, t_ctx,
             out,
             idx_s, w_s, rows_s, idx_a, w_ac, w_am, rows_a,
             idx_g, w_g, rows_g,
             idx_sml, scal_buf,
             rows_sid, buf_city, buf_lang, buf_agec, buf_gr, buf_ctx,
             acc_gen, out_buf,
             sem_a, sem_b, sem_g, sem_s, sem_o):
    wid = lax.axis_index("s") * _NC + lax.axis_index("c")
    base = wid * _RPW
    lane = lax.iota(jnp.int32, 16)
    gdn = lax.GatherDimensionNumbers(offset_dims=(), collapsed_slice_dims=(0,),
                                     start_index_map=(0,))

    def bcastl(vec, r):
        idx = jnp.bitwise_and(lane * 0 + r, 15)[:, None]
        return lax.gather(vec, idx, gdn, (1,),
                          mode=lax.GatherScatterMode.PROMISE_IN_BOUNDS)

    lanef = lane.astype(jnp.float32)
    m0 = jnp.maximum(1.0 - lanef, 0.0)
    m1 = jnp.maximum(1.0 - jnp.abs(lanef - 1.0), 0.0)

    def lane0(vec):
        idx = jnp.bitwise_and(lane, 0)[:, None]
        return lax.gather(vec, idx, gdn, (1,),
                          mode=lax.GatherScatterMode.PROMISE_IN_BOUNDS)

    def wsum_into(rows, wlist, nh, d, acc_ref, cols, stride):
        nj = d // 16
        nw = len(wlist)

        def row_body(r, carry):
            def h_body(h, accs):
                p = r * nh + h
                rs = [rows[p, pl.ds(j * 16, 16)] for j in range(nj)]
                ws = [lane0(w[pl.ds(p, 16)]) for w in wlist]
                return tuple(accs[i * nj + j] + rs[j] * ws[i]
                             for i in range(nw) for j in range(nj))

            init = tuple(jnp.zeros((16,), jnp.float32)
                         for _ in range(nw * nj))
            accs = lax.fori_loop(0, nh, h_body, init)
            for i in range(nw):
                for j in range(nj):
                    acc_ref[pl.ds(r * stride + cols[i] + j * 16, 16)] = \
                        accs[i * nj + j]
            return carry

        lax.fori_loop(0, _R, row_body, 0)

    def chunk_body(c, carry):
        row0 = base + c * _R
        rows_slice = pl.ds(row0, _R)
        hoff = row0 * _H
        goff = row0 * _GL

        pltpu.sync_copy(qh.at[pl.ds(hoff, _NH)], idx_s)
        pltpu.sync_copy(ah.at[pl.ds(hoff, _NH)], idx_a)
        song_cps = [pltpu.async_copy(t_song.at[idx_s.at[pl.ds(o, n)]],
                                     rows_s.at[pl.ds(o, n)], sem_a)
                    for o, n in _HPIECES]
        art_cps = [pltpu.async_copy(t_art.at[idx_a.at[pl.ds(o, n)]],
                                    rows_a.at[pl.ds(o, n)], sem_b)
                   for o, n in _HPIECES]

        idx_srcs = [sid, city, lang, agec, gend, rvia, sst, ssn, stype]
        idx_cps = [pltpu.async_copy(a.at[rows_slice],
                                    idx_sml.at[pl.ds(k * 16, 16)], sem_s)
                   for k, a in enumerate(idx_srcs)]
        scal_srcs = [rit, expd, agen, ten, yy, slen]
        scal_cps = [pltpu.async_copy(a.at[rows_slice],
                                     scal_buf.at[pl.ds(k * 16, 16)], sem_s)
                    for k, a in enumerate(scal_srcs)]

        pltpu.sync_copy(qc.at[pl.ds(hoff, _NH)], w_s.at[pl.ds(0, _NH)])
        pltpu.sync_copy(ac.at[pl.ds(hoff, _NH)], w_ac.at[pl.ds(0, _NH)])
        pltpu.sync_copy(am.at[pl.ds(hoff, _NH)], w_am.at[pl.ds(0, _NH)])

        pltpu.sync_copy(gid.at[pl.ds(goff, _NG)], idx_g)
        pltpu.sync_copy(glen.at[pl.ds(goff, _NG)], w_g.at[pl.ds(0, _NG)])
        genre_cp = pltpu.async_copy(t_genre.at[idx_g], rows_g, sem_g)

        for cp in idx_cps:
            cp.wait()
        cidx_gr = (idx_sml[pl.ds(64, 16)] * 10 + idx_sml[pl.ds(80, 16)])
        cidx_ctx = (idx_sml[pl.ds(96, 16)] * 375
                    + idx_sml[pl.ds(112, 16)] * 15
                    + idx_sml[pl.ds(128, 16)])
        sml_cps = [
            pltpu.async_copy(t_song.at[idx_sml.at[pl.ds(0, 16)]],
                             rows_sid, sem_g),
            pltpu.async_copy(t_city.at[idx_sml.at[pl.ds(16, 16)]],
                             buf_city, sem_g),
            pltpu.async_copy(t_lang.at[idx_sml.at[pl.ds(32, 16)]],
                             buf_lang, sem_g),
            pltpu.async_copy(t_agec.at[idx_sml.at[pl.ds(48, 16)]],
                             buf_agec, sem_g),
            pltpu.async_copy(t_gr.at[cidx_gr], buf_gr, sem_g),
            pltpu.async_copy(t_ctx.at[cidx_ctx], buf_ctx, sem_g),
        ]
        genre_cp.wait()
        wsum_into(rows_g, [w_g], _GL, 32, acc_gen, [0], 32)
        for cp in sml_cps:
            cp.wait()
        for cp in scal_cps:
            cp.wait()

        def asm_body(r, carry):
            def pair(ka, kb):
                return (bcastl(scal_buf[pl.ds(ka * 16, 16)], r) * m0
                        + bcastl(scal_buf[pl.ds(kb * 16, 16)], r) * m1)

            o = r * _OUT_D
            out_buf[pl.ds(o + 0, 16)] = buf_city[r, pl.ds(0, 16)]
            out_buf[pl.ds(o + 16, 16)] = buf_gr[r, pl.ds(0, 16)]
            out_buf[pl.ds(o + 32, 16)] = pair(0, 1)
            out_buf[pl.ds(o + 34, 16)] = buf_agec[r, pl.ds(0, 16)]
            out_buf[pl.ds(o + 42, 16)] = pair(2, 3)
            for j in range(4):
                out_buf[pl.ds(o + 236 + j * 16, 16)] = \
                    rows_sid[r, pl.ds(j * 16, 16)]
            out_buf[pl.ds(o + 300, 16)] = buf_lang[r, pl.ds(0, 16)]
            out_buf[pl.ds(o + 316, 16)] = pair(4, 5)
            out_buf[pl.ds(o + 318, 16)] = acc_gen[pl.ds(r * 32, 16)]
            out_buf[pl.ds(o + 334, 16)] = acc_gen[pl.ds(r * 32 + 16, 16)]
            out_buf[pl.ds(o + 350, 16)] = buf_ctx[r, pl.ds(0, 16)]
            out_buf[pl.ds(o + 358, 16)] = buf_ctx[r, pl.ds(8, 16)]
            return carry

        lax.fori_loop(0, _R, asm_body, 0)

        for cp in song_cps:
            cp.wait()
        wsum_into(rows_s, [w_s], _H, 64, out_buf, [44], _OUT_D)
        for cp in art_cps:
            cp.wait()
        wsum_into(rows_a, [w_ac, w_am], _H, 64, out_buf, [108, 172], _OUT_D)

        pltpu.async_copy(out_buf, out.at[pl.ds(row0 * _OUT_D, _R * _OUT_D)],
                         sem_o).wait()
        return carry

    lax.fori_loop(0, _NCHUNK, chunk_body, 0)


@jax.jit
def _run(sid, city, lang, agec, gend, rvia, sst, ssn, stype,
         rit, expd, agen, ten, yy, slen,
         qh, qc, ah, ac, am, gid, glen,
         t_song, t_art, t_genre, t_city, t_gr, t_agec, t_lang, t_ctx):
    mesh = plsc.VectorSubcoreMesh(core_axis_name="c", subcore_axis_name="s")
    k = pl.kernel(
        _sc_body,
        out_type=jax.ShapeDtypeStruct((_B * _OUT_D,), jnp.float32),
        mesh=mesh,
        compiler_params=pltpu.CompilerParams(use_tc_tiling_on_sc=False),
        scratch_types=[
            pltpu.VMEM((_NH,), jnp.int32),
            pltpu.VMEM((_NH + 16,), jnp.float32),
            pltpu.VMEM((_NH, 64), jnp.float32),
            pltpu.VMEM((_NH,), jnp.int32),
            pltpu.VMEM((_NH + 16,), jnp.float32),
            pltpu.VMEM((_NH + 16,), jnp.float32),
            pltpu.VMEM((_NH, 64), jnp.float32),
            pltpu.VMEM((_NG,), jnp.int32),
            pltpu.VMEM((_NG + 16,), jnp.float32),
            pltpu.VMEM((_NG, 32), jnp.float32),
            pltpu.VMEM((144,), jnp.int32),
            pltpu.VMEM((96,), jnp.float32),
            pltpu.VMEM((_R, 64), jnp.float32),
            pltpu.VMEM((_R, 16), jnp.float32),
            pltpu.VMEM((_R, 16), jnp.float32),
            pltpu.VMEM((_R, 16), jnp.float32),
            pltpu.VMEM((_R, 16), jnp.float32),
            pltpu.VMEM((_R, 32), jnp.float32),
            pltpu.VMEM((_R * 32,), jnp.float32),
            pltpu.VMEM((_R * _OUT_D,), jnp.float32),
            pltpu.SemaphoreType.DMA,
            pltpu.SemaphoreType.DMA,
            pltpu.SemaphoreType.DMA,
            pltpu.SemaphoreType.DMA,
            pltpu.SemaphoreType.DMA,
        ],
    )
    return k(sid, city, lang, agec, gend, rvia, sst, ssn, stype,
             rit, expd, agen, ten, yy, slen,
             qh, qc, ah, ac, am, gid, glen,
             t_song, t_art, t_genre, t_city, t_gr, t_agec, t_lang, t_ctx)


def kernel(city, gender, registered_via, msno_age_catg, registration_init_time,
           expiration_date, msno_age_num, msno_tenure,
           msno_pos_query_hist, msno_pos_query_count,
           msno_artist_name_hist, msno_artist_name_count, msno_artist_name_mean,
           song_id, language, song_yy, song_length,
           genre_ids, song_genre_ids_len,
           source_system_tab, source_screen_name, source_type,
           emb_song, emb_artist, emb_genre, emb_city, emb_gender,
           emb_registered_via, emb_msno_age_catg, emb_language,
           emb_sst, emb_ssn, emb_stype):
    t_gr = jnp.concatenate([jnp.repeat(emb_gender, 10, axis=0),
                            jnp.tile(emb_registered_via, (3, 1))], axis=1)
    z = jnp.zeros((10, 25, 15, 8), jnp.float32)
    t_ctx = jnp.concatenate([
        jnp.broadcast_to(emb_sst[:, None, None, :], (10, 25, 15, 8)),
        jnp.broadcast_to(emb_ssn[None, :, None, :], (10, 25, 15, 8)),
        jnp.broadcast_to(emb_stype[None, None, :, :], (10, 25, 15, 8)),
        z], axis=3).reshape(3750, 32)
    t_agec = jnp.pad(emb_msno_age_catg, ((0, 0), (0, 8)))
    flat = _run(song_id, city, language, msno_age_catg,
                gender, registered_via,
                source_system_tab, source_screen_name, source_type,
                registration_init_time, expiration_date,
                msno_age_num, msno_tenure, song_yy, song_length,
                msno_pos_query_hist.reshape(-1), msno_pos_query_count.reshape(-1),
                msno_artist_name_hist.reshape(-1),
                msno_artist_name_count.reshape(-1),
                msno_artist_name_mean.reshape(-1),
                genre_ids.reshape(-1), song_genre_ids_len.reshape(-1),
                emb_song, emb_artist, emb_genre, emb_city, t_gr, t_agec,
                emb_language, t_ctx)
    return flat.reshape(_B, _OUT_D)


# speedup vs baseline: 1.1173x; 1.0100x over previous
import jax
import jax.numpy as jnp
from jax import lax
from jax.experimental import pallas as pl
from jax.experimental.pallas import tpu as pltpu
from jax.experimental.pallas import tpu_sc as plsc

_B = 16384
_H = 50
_GL = 8
_NC = 2
_NS = 16
_NW = _NC * _NS
_RPW = _B // _NW
_R = 16
_NCHUNK = _RPW // _R

_NH = _R * _H
_NG = _R * _GL

_HPIECES = [(0, 128), (128, 128), (256, 128), (384, 128),
            (512, 128), (640, 128), (768, 32)]

_OUT_D = 374


def _sc_body(sid, city, lang, agec, gend, rvia, sst, ssn, stype,
             rit, expd, agen, ten, yy, slen,
             qh, qc, ah, ac, am, gid, glen,
             t_song, t_art, t_genre, t_city, t_gr, t_agec, t_lang, t_ctx,
             out,
             idx_s, w_s, rows_s, idx_a, w_ac, w_am, rows_a,
             idx_g, w_g, rows_g,
             idx_sml, scal_buf,
             rows_sid, buf_city, buf_lang, buf_agec, buf_gr, buf_ctx,
             acc_gen, out_buf,
             sem_a, sem_b, sem_g, sem_s, sem_o):
    wid = lax.axis_index("s") * _NC + lax.axis_index("c")
    base = wid * _RPW
    lane = lax.iota(jnp.int32, 16)
    gdn = lax.GatherDimensionNumbers(offset_dims=(), collapsed_slice_dims=(0,),
                                     start_index_map=(0,))

    def bcastl(vec, r):
        idx = jnp.bitwise_and(lane * 0 + r, 15)[:, None]
        return lax.gather(vec, idx, gdn, (1,),
                          mode=lax.GatherScatterMode.PROMISE_IN_BOUNDS)

    lanef = lane.astype(jnp.float32)
    m0 = jnp.maximum(1.0 - lanef, 0.0)
    m1 = jnp.maximum(1.0 - jnp.abs(lanef - 1.0), 0.0)

    def lane0(vec):
        idx = jnp.bitwise_and(lane, 0)[:, None]
        return lax.gather(vec, idx, gdn, (1,),
                          mode=lax.GatherScatterMode.PROMISE_IN_BOUNDS)

    def wsum_into(rows, wlist, nh, d, acc_ref, cols, stride, unroll):
        nj = d // 16
        nw = len(wlist)

        def row_body(r, carry):
            rp = r * nh

            def h_body(h0, accs):
                accs = list(accs)
                for k in range(unroll):
                    p = rp + h0 * unroll + k
                    rs = [rows[p, pl.ds(j * 16, 16)] for j in range(nj)]
                    ws = [lane0(w[pl.ds(p, 16)]) for w in wlist]
                    accs = [accs[i * nj + j] + rs[j] * ws[i]
                            for i in range(nw) for j in range(nj)]
                return tuple(accs)

            init = tuple(jnp.zeros((16,), jnp.float32)
                         for _ in range(nw * nj))
            accs = lax.fori_loop(0, nh // unroll, h_body, init)
            for i in range(nw):
                for j in range(nj):
                    acc_ref[pl.ds(r * stride + cols[i] + j * 16, 16)] = \
                        accs[i * nj + j]
            return carry

        lax.fori_loop(0, _R, row_body, 0)

    def chunk_body(c, carry):
        row0 = base + c * _R
        rows_slice = pl.ds(row0, _R)
        hoff = row0 * _H
        goff = row0 * _GL

        pltpu.sync_copy(qh.at[pl.ds(hoff, _NH)], idx_s)
        pltpu.sync_copy(ah.at[pl.ds(hoff, _NH)], idx_a)
        song_cps = [pltpu.async_copy(t_song.at[idx_s.at[pl.ds(o, n)]],
                                     rows_s.at[pl.ds(o, n)], sem_a)
                    for o, n in _HPIECES]
        art_cps = [pltpu.async_copy(t_art.at[idx_a.at[pl.ds(o, n)]],
                                    rows_a.at[pl.ds(o, n)], sem_b)
                   for o, n in _HPIECES]

        idx_srcs = [sid, city, lang, agec, gend, rvia, sst, ssn, stype]
        idx_cps = [pltpu.async_copy(a.at[rows_slice],
                                    idx_sml.at[pl.ds(k * 16, 16)], sem_s)
                   for k, a in enumerate(idx_srcs)]
        scal_srcs = [rit, expd, agen, ten, yy, slen]
        scal_cps = [pltpu.async_copy(a.at[rows_slice],
                                     scal_buf.at[pl.ds(k * 16, 16)], sem_s)
                    for k, a in enumerate(scal_srcs)]

        pltpu.sync_copy(qc.at[pl.ds(hoff, _NH)], w_s.at[pl.ds(0, _NH)])
        pltpu.sync_copy(ac.at[pl.ds(hoff, _NH)], w_ac.at[pl.ds(0, _NH)])
        pltpu.sync_copy(am.at[pl.ds(hoff, _NH)], w_am.at[pl.ds(0, _NH)])

        pltpu.sync_copy(gid.at[pl.ds(goff, _NG)], idx_g)
        pltpu.sync_copy(glen.at[pl.ds(goff, _NG)], w_g.at[pl.ds(0, _NG)])
        genre_cp = pltpu.async_copy(t_genre.at[idx_g], rows_g, sem_g)

        for cp in idx_cps:
            cp.wait()
        cidx_gr = (idx_sml[pl.ds(64, 16)] * 10 + idx_sml[pl.ds(80, 16)])
        cidx_ctx = (idx_sml[pl.ds(96, 16)] * 375
                    + idx_sml[pl.ds(112, 16)] * 15
                    + idx_sml[pl.ds(128, 16)])
        sml_cps = [
            pltpu.async_copy(t_song.at[idx_sml.at[pl.ds(0, 16)]],
                             rows_sid, sem_g),
            pltpu.async_copy(t_city.at[idx_sml.at[pl.ds(16, 16)]],
                             buf_city, sem_g),
            pltpu.async_copy(t_lang.at[idx_sml.at[pl.ds(32, 16)]],
                             buf_lang, sem_g),
            pltpu.async_copy(t_agec.at[idx_sml.at[pl.ds(48, 16)]],
                             buf_agec, sem_g),
            pltpu.async_copy(t_gr.at[cidx_gr], buf_gr, sem_g),
            pltpu.async_copy(t_ctx.at[cidx_ctx], buf_ctx, sem_g),
        ]
        genre_cp.wait()
        wsum_into(rows_g, [w_g], _GL, 32, acc_gen, [0], 32, 4)
        for cp in sml_cps:
            cp.wait()
        for cp in scal_cps:
            cp.wait()

        def asm_body(r, carry):
            def pair(ka, kb):
                return (bcastl(scal_buf[pl.ds(ka * 16, 16)], r) * m0
                        + bcastl(scal_buf[pl.ds(kb * 16, 16)], r) * m1)

            o = r * _OUT_D
            out_buf[pl.ds(o + 0, 16)] = buf_city[r, pl.ds(0, 16)]
            out_buf[pl.ds(o + 16, 16)] = buf_gr[r, pl.ds(0, 16)]
            out_buf[pl.ds(o + 32, 16)] = pair(0, 1)
            out_buf[pl.ds(o + 34, 16)] = buf_agec[r, pl.ds(0, 16)]
            out_buf[pl.ds(o + 42, 16)] = pair(2, 3)
            for j in range(4):
                out_buf[pl.ds(o + 236 + j * 16, 16)] = \
                    rows_sid[r, pl.ds(j * 16, 16)]
            out_buf[pl.ds(o + 300, 16)] = buf_lang[r, pl.ds(0, 16)]
            out_buf[pl.ds(o + 316, 16)] = pair(4, 5)
            out_buf[pl.ds(o + 318, 16)] = acc_gen[pl.ds(r * 32, 16)]
            out_buf[pl.ds(o + 334, 16)] = acc_gen[pl.ds(r * 32 + 16, 16)]
            out_buf[pl.ds(o + 350, 16)] = buf_ctx[r, pl.ds(0, 16)]
            out_buf[pl.ds(o + 358, 16)] = buf_ctx[r, pl.ds(8, 16)]
            return carry

        lax.fori_loop(0, _R, asm_body, 0)

        for cp in song_cps:
            cp.wait()
        wsum_into(rows_s, [w_s], _H, 64, out_buf, [44], _OUT_D, 5)
        for cp in art_cps:
            cp.wait()
        wsum_into(rows_a, [w_ac, w_am], _H, 64, out_buf, [108, 172],
                  _OUT_D, 5)

        pltpu.async_copy(out_buf, out.at[pl.ds(row0 * _OUT_D, _R * _OUT_D)],
                         sem_o).wait()
        return carry

    lax.fori_loop(0, _NCHUNK, chunk_body, 0)


@jax.jit
def _run(sid, city, lang, agec, gend, rvia, sst, ssn, stype,
         rit, expd, agen, ten, yy, slen,
         qh, qc, ah, ac, am, gid, glen,
         t_song, t_art, t_genre, t_city, t_gr, t_agec, t_lang, t_ctx):
    mesh = plsc.VectorSubcoreMesh(core_axis_name="c", subcore_axis_name="s")
    k = pl.kernel(
        _sc_body,
        out_type=jax.ShapeDtypeStruct((_B * _OUT_D,), jnp.float32),
        mesh=mesh,
        compiler_params=pltpu.CompilerParams(use_tc_tiling_on_sc=False),
        scratch_types=[
            pltpu.VMEM((_NH,), jnp.int32),
            pltpu.VMEM((_NH + 16,), jnp.float32),
            pltpu.VMEM((_NH, 64), jnp.float32),
            pltpu.VMEM((_NH,), jnp.int32),
            pltpu.VMEM((_NH + 16,), jnp.float32),
            pltpu.VMEM((_NH + 16,), jnp.float32),
            pltpu.VMEM((_NH, 64), jnp.float32),
            pltpu.VMEM((_NG,), jnp.int32),
            pltpu.VMEM((_NG + 16,), jnp.float32),
            pltpu.VMEM((_NG, 32), jnp.float32),
            pltpu.VMEM((144,), jnp.int32),
            pltpu.VMEM((96,), jnp.float32),
            pltpu.VMEM((_R, 64), jnp.float32),
            pltpu.VMEM((_R, 16), jnp.float32),
            pltpu.VMEM((_R, 16), jnp.float32),
            pltpu.VMEM((_R, 16), jnp.float32),
            pltpu.VMEM((_R, 16), jnp.float32),
            pltpu.VMEM((_R, 32), jnp.float32),
            pltpu.VMEM((_R * 32,), jnp.float32),
            pltpu.VMEM((_R * _OUT_D,), jnp.float32),
            pltpu.SemaphoreType.DMA,
            pltpu.SemaphoreType.DMA,
            pltpu.SemaphoreType.DMA,
            pltpu.SemaphoreType.DMA,
            pltpu.SemaphoreType.DMA,
        ],
    )
    return k(sid, city, lang, agec, gend, rvia, sst, ssn, stype,
             rit, expd, agen, ten, yy, slen,
             qh, qc, ah, ac, am, gid, glen,
             t_song, t_art, t_genre, t_city, t_gr, t_agec, t_lang, t_ctx)


def kernel(city, gender, registered_via, msno_age_catg, registration_init_time,
           expiration_date, msno_age_num, msno_tenure,
           msno_pos_query_hist, msno_pos_query_count,
           msno_artist_name_hist, msno_artist_name_count, msno_artist_name_mean,
           song_id, language, song_yy, song_length,
           genre_ids, song_genre_ids_len,
           source_system_tab, source_screen_name, source_type,
           emb_song, emb_artist, emb_genre, emb_city, emb_gender,
           emb_registered_via, emb_msno_age_catg, emb_language,
           emb_sst, emb_ssn, emb_stype):
    t_gr = jnp.concatenate([jnp.repeat(emb_gender, 10, axis=0),
                            jnp.tile(emb_registered_via, (3, 1))], axis=1)
    z = jnp.zeros((10, 25, 15, 8), jnp.float32)
    t_ctx = jnp.concatenate([
        jnp.broadcast_to(emb_sst[:, None, None, :], (10, 25, 15, 8)),
        jnp.broadcast_to(emb_ssn[None, :, None, :], (10, 25, 15, 8)),
        jnp.broadcast_to(emb_stype[None, None, :, :], (10, 25, 15, 8)),
        z], axis=3).reshape(3750, 32)
    t_agec = jnp.pad(emb_msno_age_catg, ((0, 0), (0, 8)))
    flat = _run(song_id, city, language, msno_age_catg,
                gender, registered_via,
                source_system_tab, source_screen_name, source_type,
                registration_init_time, expiration_date,
                msno_age_num, msno_tenure, song_yy, song_length,
                msno_pos_query_hist.reshape(-1), msno_pos_query_count.reshape(-1),
                msno_artist_name_hist.reshape(-1),
                msno_artist_name_count.reshape(-1),
                msno_artist_name_mean.reshape(-1),
                genre_ids.reshape(-1), song_genre_ids_len.reshape(-1),
                emb_song, emb_artist, emb_genre, emb_city, t_gr, t_agec,
                emb_language, t_ctx)
    return flat.reshape(_B, _OUT_D)
